# baseline (speedup 1.0000x reference)
import functools

import jax
import jax.numpy as jnp
from jax import lax
from jax.experimental import pallas as pl
from jax.experimental.pallas import tpu as pltpu
from jax.experimental.pallas import tpu_sc as plsc

N = 16384
C = 1000
L = 16
NC, NS = 2, 16
NW = NC * NS
BPW = N // NW
GBLK = 128
NGATH = BPW // GBLK

_LN2 = 0.6931471805599453
_SQRT2 = 1.4142135623730951


def _vlog(x):
    bits = lax.bitcast_convert_type(x, jnp.int32)
    e = lax.shift_right_logical(bits, 23) - 127
    m = lax.bitcast_convert_type((bits & 0x007FFFFF) | 0x3F800000, jnp.float32)
    big = m > _SQRT2
    m = jnp.where(big, m * 0.5, m)
    e = e + jnp.where(big, 1, 0)
    s = (m - 1.0) / (m + 1.0)
    z = s * s
    p = 1.0 + z * (1 / 3 + z * (1 / 5 + z * (1 / 7 + z * (1 / 9))))
    return e.astype(jnp.float32) * _LN2 + 2.0 * s * p


_MESH = plsc.VectorSubcoreMesh(core_axis_name="c", subcore_axis_name="s")


@functools.partial(
    pl.kernel,
    mesh=_MESH,
    out_type=jax.ShapeDtypeStruct((NW, L), jnp.float32),
    scratch_types=[
        pltpu.VMEM((BPW,), jnp.int32),
        pltpu.VMEM((NGATH, GBLK), jnp.int32),
        pltpu.VMEM((GBLK,), jnp.float32),
        pltpu.VMEM((L,), jnp.float32),
        pltpu.SemaphoreType.DMA,
    ],
)
def _nll_partials(flat_hbm, tgt_hbm, out_hbm, tgt_v, idx_v, val_v, acc_v, sem):
    wid = lax.axis_index("s") * NC + lax.axis_index("c")
    base = wid * BPW
    pltpu.sync_copy(tgt_hbm.at[pl.ds(base, BPW)], tgt_v)
    lane = lax.iota(jnp.int32, L)
    per = GBLK // L
    for j in range(BPW // L):
        t = tgt_v[pl.ds(j * L, L)]
        idx_v[j // per, pl.ds((j % per) * L, L)] = (base + j * L) * C + lane * C + t
    acc = jnp.zeros((L,), jnp.float32)
    for k in range(NGATH):
        pltpu.async_copy(flat_hbm.at[idx_v.at[k]], val_v, sem).wait()
        for m in range(per):
            acc = acc + _vlog(val_v[pl.ds(m * L, L)])
    acc_v[...] = acc
    pltpu.sync_copy(acc_v, out_hbm.at[wid])


def kernel(input, target):
    flat = input.reshape(N * C)
    partials = _nll_partials(flat, target.astype(jnp.int32))
    return -jnp.sum(partials) / jnp.float32(N)


# speedup vs baseline: 1.7136x; 1.7136x over previous
import functools

import jax
import jax.numpy as jnp
from jax import lax
from jax.experimental import pallas as pl
from jax.experimental.pallas import tpu as pltpu
from jax.experimental.pallas import tpu_sc as plsc

N = 16384
C = 1000
L = 16
NC, NS = 2, 16
NW = NC * NS
BPW = N // NW
NCH = BPW // L
NB = 8
BCAP = 640
MAXCH = NCH + NB

_LN2 = 0.6931471805599453
_SQRT2 = 1.4142135623730951


def _vlog(x):
    bits = lax.bitcast_convert_type(x, jnp.int32)
    e = lax.shift_right_logical(bits, 23) - 127
    m = lax.bitcast_convert_type((bits & 0x007FFFFF) | 0x3F800000, jnp.float32)
    big = m > _SQRT2
    m = jnp.where(big, m * 0.5, m)
    e = e + jnp.where(big, 1, 0)
    s = (m - 1.0) / (m + 1.0)
    z = s * s
    p = 1.0 + z * (1 / 3 + z * (1 / 5 + z * (1 / 7 + z * (1 / 9))))
    return e.astype(jnp.float32) * _LN2 + 2.0 * s * p


_MESH = plsc.VectorSubcoreMesh(core_axis_name="c", subcore_axis_name="s")


@functools.partial(
    pl.kernel,
    mesh=_MESH,
    out_type=jax.ShapeDtypeStruct((NW, L), jnp.float32),
    compiler_params=pltpu.CompilerParams(needs_layout_passes=False),
    scratch_types=[
        pltpu.VMEM((BPW,), jnp.int32),
        pltpu.VMEM((NB, BCAP), jnp.int32),
        pltpu.VMEM((NB, BCAP), jnp.int32),
        pltpu.VMEM((MAXCH * L, 128), jnp.float32),
        pltpu.VMEM((L,), jnp.float32),
        pltpu.SemaphoreType.DMA,
    ],
)
def _nll_partials(table_hbm, tgt_hbm, out_hbm, tgt_v, rows_b, cols_b, win_v,
                  acc_v, sem):
    wid = lax.axis_index("s") * NC + lax.axis_index("c")
    base = wid * BPW
    pltpu.sync_copy(tgt_hbm.at[pl.ds(base, BPW)], tgt_v)
    lane = lax.iota(jnp.int32, L)

    offs = [jnp.int32(0)] * NB
    for j in range(NCH):
        t = tgt_v[pl.ds(j * L, L)]
        rows = base + j * L + lane
        kvec = lax.shift_right_logical(t, 7)
        col = t & 127
        for k in range(NB):
            msk = kvec == k
            plsc.store_compressed(rows_b.at[k, pl.ds(offs[k], L)], rows, mask=msk)
            plsc.store_compressed(cols_b.at[k, pl.ds(offs[k], L)], col, mask=msk)
            offs[k] = offs[k] + jnp.sum(msk.astype(jnp.int32))

    for k in range(NB):
        rows_b[k, pl.ds(offs[k], L)] = base + lane
        cols_b[k, pl.ds(offs[k], L)] = lane

    nchunks = [lax.shift_right_logical(offs[k] + (L - 1), 4) for k in range(NB)]
    gptr = jnp.int32(0)
    for k in range(NB):
        def _fire(cc, gp, k=k):
            start = wid * 0 + k * 128
            pltpu.async_copy(
                table_hbm.at[rows_b.at[k, pl.ds(cc * L, L)],
                             pl.ds(start, 128)],
                win_v.at[pl.ds(gp * L, L)], sem)
            return gp + 1
        gptr = lax.fori_loop(0, nchunks[k], _fire, gptr)

    def _drain(cc, carry):
        pltpu.make_async_copy(
            table_hbm.at[pl.ds(0, L), pl.ds(0, 128)],
            win_v.at[pl.ds(0, L)], sem).wait()
        return carry
    lax.fori_loop(0, gptr, _drain, jnp.int32(0))

    acc = jnp.zeros((L,), jnp.float32)
    gptr = jnp.int32(0)
    for k in range(NB):
        def _pick(cc, carry, k=k):
            a, gp = carry
            cols = cols_b[k, pl.ds(cc * L, L)]
            vals = plsc.load_gather(win_v, [gp * L + lane, cols])
            valid = (cc * L + lane) < offs[k]
            a = a + jnp.where(valid, _vlog(vals), 0.0)
            return a, gp + 1
        acc, gptr = lax.fori_loop(0, nchunks[k], _pick, (acc, gptr))

    acc_v[...] = acc
    pltpu.sync_copy(acc_v, out_hbm.at[wid])


def kernel(input, target):
    partials = _nll_partials(input, target.astype(jnp.int32))
    return -jnp.sum(partials) / jnp.float32(N)
